# baseline (speedup 1.0000x reference)
import functools

import jax
import jax.numpy as jnp
from jax import lax
from jax.experimental import pallas as pl
from jax.experimental.pallas import tpu as pltpu
from jax.experimental.pallas import tpu_sc as plsc

E = 16
DIM = 1024
INTER = 512
T = 32
A = 2

_NC = 2
_NS = 16
_B = T * A
_BPW = 16
_NW = _B // _BPW


def _moe_dense_kernel(x_ref, w1_ref, w2_ref, out_ref):
    x = x_ref[...]
    h = jnp.dot(x, w1_ref[0], preferred_element_type=jnp.float32)
    gate = h[:, :INTER]
    up = h[:, INTER:]
    act = (gate * jax.nn.sigmoid(gate)) * up
    out_ref[0] = jnp.dot(act, w2_ref[0], preferred_element_type=jnp.float32)


_sc_mesh = plsc.VectorSubcoreMesh(core_axis_name="c", subcore_axis_name="s")


@functools.partial(
    pl.kernel,
    mesh=_sc_mesh,
    out_type=jax.ShapeDtypeStruct((_B, DIM), jnp.float32),
    scratch_types=[
        pltpu.VMEM((_BPW,), jnp.int32),
        pltpu.VMEM((_BPW,), jnp.int32),
        pltpu.VMEM((_BPW, DIM), jnp.float32),
        pltpu.SemaphoreType.DMA,
    ],
)
def _sc_route_gather(table_hbm, ei_hbm, out_hbm, ei_v, idx_v, rows_v, sem):
    wid = lax.axis_index("s") * _NC + lax.axis_index("c")

    @pl.when(wid < _NW)
    def _():
        base = wid * _BPW
        pltpu.sync_copy(ei_hbm.at[pl.ds(base, _BPW)], ei_v)
        ei = ei_v[...]
        b = base + lax.iota(jnp.int32, _BPW)
        tok = lax.shift_right_logical(b, 1)
        idx_v[...] = ei * T + tok
        pltpu.async_copy(table_hbm.at[idx_v], rows_v, sem).wait()
        pltpu.sync_copy(rows_v, out_hbm.at[pl.ds(base, _BPW)])


def kernel(x, expert_indices, w1, w2):
    dense = pl.pallas_call(
        _moe_dense_kernel,
        grid=(E,),
        in_specs=[
            pl.BlockSpec((T, DIM), lambda e: (0, 0)),
            pl.BlockSpec((1, DIM, 2 * INTER), lambda e: (e, 0, 0)),
            pl.BlockSpec((1, INTER, DIM), lambda e: (e, 0, 0)),
        ],
        out_specs=pl.BlockSpec((1, T, DIM), lambda e: (e, 0, 0)),
        out_shape=jax.ShapeDtypeStruct((E, T, DIM), jnp.float32),
    )(x, w1, w2)

    table = dense.reshape(E * T, DIM)
    ei_flat = expert_indices.astype(jnp.int32).reshape(_B)
    out = _sc_route_gather(table, ei_flat)
    return out.reshape(T, A, DIM)


# speedup vs baseline: 1.5208x; 1.5208x over previous
import jax
import jax.numpy as jnp
from jax.experimental import pallas as pl

E = 16
DIM = 1024
INTER = 512
T = 32
A = 2


def _moe_ffn_kernel(idx_ref, x_ref, w1_ref, w2_ref, out0_ref, out1_ref):
    e = pl.program_id(0)
    x = x_ref[...].astype(jnp.bfloat16)
    h = jnp.dot(x, w1_ref[0].astype(jnp.bfloat16),
                preferred_element_type=jnp.float32)
    gate = h[:, :INTER]
    up = h[:, INTER:]
    act = (gate * jax.nn.sigmoid(gate)) * up
    out_e = jnp.dot(act.astype(jnp.bfloat16), w2_ref[0].astype(jnp.bfloat16),
                    preferred_element_type=jnp.float32)

    mask = idx_ref[...] == e
    c0 = jnp.where(mask[:, 0:1], out_e, 0.0)
    c1 = jnp.where(mask[:, 1:2], out_e, 0.0)

    @pl.when(e == 0)
    def _init():
        out0_ref[...] = c0
        out1_ref[...] = c1

    @pl.when(e != 0)
    def _accum():
        out0_ref[...] += c0
        out1_ref[...] += c1


def kernel(x, expert_indices, w1, w2):
    expert_indices = expert_indices.astype(jnp.int32)
    out0, out1 = pl.pallas_call(
        _moe_ffn_kernel,
        grid=(E,),
        in_specs=[
            pl.BlockSpec((T, A), lambda e: (0, 0)),
            pl.BlockSpec((T, DIM), lambda e: (0, 0)),
            pl.BlockSpec((1, DIM, 2 * INTER), lambda e: (e, 0, 0)),
            pl.BlockSpec((1, INTER, DIM), lambda e: (e, 0, 0)),
        ],
        out_specs=[
            pl.BlockSpec((T, DIM), lambda e: (0, 0)),
            pl.BlockSpec((T, DIM), lambda e: (0, 0)),
        ],
        out_shape=[
            jax.ShapeDtypeStruct((T, DIM), jnp.float32),
            jax.ShapeDtypeStruct((T, DIM), jnp.float32),
        ],
    )(expert_indices, x, w1, w2)
    return jnp.stack([out0, out1], axis=1)


# speedup vs baseline: 1.5282x; 1.0049x over previous
import jax
import jax.numpy as jnp
from jax.experimental import pallas as pl
from jax.experimental.pallas import tpu as pltpu

E = 16
DIM = 1024
INTER = 512
T = 32
A = 2


def _moe_ffn_kernel(idx_ref, x_ref, w1_ref, w2_ref, out0_ref, out1_ref):
    e = pl.program_id(0)
    x = x_ref[...]
    h = jnp.dot(x, w1_ref[0], preferred_element_type=jnp.float32)
    gate = h[:, :INTER]
    up = h[:, INTER:]
    act = (gate * jax.nn.sigmoid(gate)) * up
    out_e = jnp.dot(act, w2_ref[0], preferred_element_type=jnp.float32)

    mask = idx_ref[...] == e
    c0 = jnp.where(mask[:, 0:1], out_e, 0.0)
    c1 = jnp.where(mask[:, 1:2], out_e, 0.0)

    @pl.when(e == 0)
    def _init():
        out0_ref[...] = c0
        out1_ref[...] = c1

    @pl.when(e != 0)
    def _accum():
        out0_ref[...] += c0
        out1_ref[...] += c1


def kernel(x, expert_indices, w1, w2):
    expert_indices = expert_indices.astype(jnp.int32)
    out0, out1 = pl.pallas_call(
        _moe_ffn_kernel,
        grid=(E,),
        in_specs=[
            pl.BlockSpec((T, A), lambda e: (0, 0)),
            pl.BlockSpec((T, DIM), lambda e: (0, 0)),
            pl.BlockSpec((1, DIM, 2 * INTER), lambda e: (e, 0, 0)),
            pl.BlockSpec((1, INTER, DIM), lambda e: (e, 0, 0)),
        ],
        out_specs=[
            pl.BlockSpec((T, DIM), lambda e: (0, 0)),
            pl.BlockSpec((T, DIM), lambda e: (0, 0)),
        ],
        out_shape=[
            jax.ShapeDtypeStruct((T, DIM), jnp.float32),
            jax.ShapeDtypeStruct((T, DIM), jnp.float32),
        ],
        compiler_params=pltpu.CompilerParams(vmem_limit_bytes=100 * 1024 * 1024),
    )(expert_indices, x, w1, w2)
    return jnp.stack([out0, out1], axis=1)


# speedup vs baseline: 1.5335x; 1.0034x over previous
import jax
import jax.numpy as jnp
from jax.experimental import pallas as pl
from jax.experimental.pallas import tpu as pltpu

E = 16
DIM = 1024
INTER = 512
T = 32
A = 2
HALF = DIM // 2


def _moe_ffn_kernel(idx_ref, x_ref, w1a_ref, w1b_ref, w2_ref, out0_ref, out1_ref):
    e = pl.program_id(0)
    x = x_ref[...]
    h = (jnp.dot(x[:, :HALF], w1a_ref[0], preferred_element_type=jnp.float32)
         + jnp.dot(x[:, HALF:], w1b_ref[0], preferred_element_type=jnp.float32))
    gate = h[:, :INTER]
    up = h[:, INTER:]
    act = (gate * jax.nn.sigmoid(gate)) * up
    out_e = jnp.dot(act, w2_ref[0], preferred_element_type=jnp.float32)

    mask = idx_ref[...] == e
    c0 = jnp.where(mask[:, 0:1], out_e, 0.0)
    c1 = jnp.where(mask[:, 1:2], out_e, 0.0)

    @pl.when(e == 0)
    def _init():
        out0_ref[...] = c0
        out1_ref[...] = c1

    @pl.when(e != 0)
    def _accum():
        out0_ref[...] += c0
        out1_ref[...] += c1


def kernel(x, expert_indices, w1, w2):
    expert_indices = expert_indices.astype(jnp.int32)
    out0, out1 = pl.pallas_call(
        _moe_ffn_kernel,
        grid=(E,),
        in_specs=[
            pl.BlockSpec((T, A), lambda e: (0, 0)),
            pl.BlockSpec((T, DIM), lambda e: (0, 0)),
            pl.BlockSpec((1, HALF, 2 * INTER), lambda e: (e, 0, 0)),
            pl.BlockSpec((1, HALF, 2 * INTER), lambda e: (e, 1, 0)),
            pl.BlockSpec((1, INTER, DIM), lambda e: (e, 0, 0)),
        ],
        out_specs=[
            pl.BlockSpec((T, DIM), lambda e: (0, 0)),
            pl.BlockSpec((T, DIM), lambda e: (0, 0)),
        ],
        out_shape=[
            jax.ShapeDtypeStruct((T, DIM), jnp.float32),
            jax.ShapeDtypeStruct((T, DIM), jnp.float32),
        ],
        compiler_params=pltpu.CompilerParams(vmem_limit_bytes=100 * 1024 * 1024),
    )(expert_indices, x, w1, w1, w2)
    return jnp.stack([out0, out1], axis=1)
